# Initial kernel scaffold; baseline (speedup 1.0000x reference)
import jax
import jax.numpy as jnp
from jax.experimental import pallas as pl


def kernel(x, indices):
    raise NotImplementedError("write your pallas kernel here")


# speedup vs baseline: 4.0059x; 4.0059x over previous
import functools

import jax
import jax.numpy as jnp
from jax import lax
from jax.experimental import pallas as pl
from jax.experimental.pallas import tpu as pltpu
from jax.experimental.pallas import tpu_sc as plsc

P = 384
N = 36864
HW = 147456
HALF = HW // 2
CH = 9216
N_WORKERS = 32
TASKS_PER_WORKER = (P * 2) // N_WORKERS
L = 16


def _unpool_body(sidx_hbm, sval_hbm, out_hbm, out_buf, idx_buf, x_buf):
    wid = lax.axis_index("s") * 2 + lax.axis_index("c")

    def task_body(t, carry):
        tid = wid * TASKS_PER_WORKER + t
        p = tid // 2
        h = tid % 2
        base = p * HW + h * HALF

        def zero_body(i, c):
            out_buf[pl.ds(i * L, L)] = jnp.zeros((L,), jnp.float32)
            return c

        lax.fori_loop(0, HALF // L, zero_body, 0)

        def chunk_body(c, carry2):
            start = p * N + c * CH
            is_last = start + CH >= P * N

            @pl.when(is_last)
            def _():
                pltpu.sync_copy(sidx_hbm.at[pl.ds(start, CH)],
                                idx_buf.at[pl.ds(0, CH)])
                idx_buf[pl.ds(CH, L)] = jnp.full((L,), -1, jnp.int32)

            @pl.when(jnp.logical_not(is_last))
            def _():
                pltpu.sync_copy(sidx_hbm.at[pl.ds(start, CH + L)], idx_buf)

            pltpu.sync_copy(sval_hbm.at[pl.ds(start, CH)], x_buf)

            def vec_body(i, c3):
                kv = idx_buf[pl.ds(i * L, L)]
                nv = idx_buf[pl.ds(i * L + 1, L)]
                xv = x_buf[pl.ds(i * L, L)]
                local = kv - base
                m = (local >= 0) & (local < HALF) & (kv != nv)
                safe = jnp.where(m, local, 0)
                plsc.store_scatter(out_buf, [safe], xv, mask=m)
                return c3

            lax.fori_loop(0, CH // L, vec_body, 0)
            return carry2

        lax.fori_loop(0, N // CH, chunk_body, 0)
        pltpu.sync_copy(out_buf, out_hbm.at[p, pl.ds(h * HALF, HALF)])
        return carry

    lax.fori_loop(0, TASKS_PER_WORKER, task_body, 0)


_unpool = functools.partial(
    pl.kernel,
    out_type=jax.ShapeDtypeStruct((P, HW), jnp.float32),
    mesh=plsc.VectorSubcoreMesh(core_axis_name="c", subcore_axis_name="s"),
    compiler_params=pltpu.CompilerParams(needs_layout_passes=False),
    scratch_types=[
        pltpu.VMEM((HALF,), jnp.float32),
        pltpu.VMEM((CH + L,), jnp.int32),
        pltpu.VMEM((CH,), jnp.float32),
    ],
)(_unpool_body)


def kernel(x, indices):
    b, c, h, w = x.shape
    plane = jnp.arange(b * c, dtype=jnp.int32)[:, None] * HW
    gidx = (indices.reshape(b * c, h * w) + plane).reshape(-1)
    s_idx, s_val = lax.sort((gidx, x.reshape(-1)), dimension=0,
                            is_stable=False, num_keys=1)
    out = _unpool(s_idx, s_val)
    return out.reshape(b, c, 2 * h, 2 * w)


# speedup vs baseline: 4.0596x; 1.0134x over previous
import functools

import jax
import jax.numpy as jnp
from jax import lax
from jax.experimental import pallas as pl
from jax.experimental.pallas import tpu as pltpu
from jax.experimental.pallas import tpu_sc as plsc

P = 384
N = 36864
HW = 147456
HALF = HW // 2
CH = 9216
N_WORKERS = 32
TASKS_PER_WORKER = (P * 2) // N_WORKERS
L = 16


def _unpool_body(sidx_hbm, sval_hbm, out_hbm, out_buf, idx_buf, x_buf):
    wid = lax.axis_index("s") * 2 + lax.axis_index("c")

    def task_body(t, carry):
        tid = wid * TASKS_PER_WORKER + t
        p = tid // 2
        h = tid % 2
        base = p * HW + h * HALF

        def zero_body(i, c):
            out_buf[pl.ds(i * L, L)] = jnp.zeros((L,), jnp.float32)
            return c

        lax.fori_loop(0, HALF // L, zero_body, 0)

        def chunk_body(c, carry2):
            start = p * N + c * CH
            is_last = start + CH >= P * N

            @pl.when(is_last)
            def _():
                pltpu.sync_copy(sidx_hbm.at[pl.ds(start, CH)],
                                idx_buf.at[pl.ds(0, CH)])
                idx_buf[pl.ds(CH, L)] = jnp.full((L,), -1, jnp.int32)

            @pl.when(jnp.logical_not(is_last))
            def _():
                pltpu.sync_copy(sidx_hbm.at[pl.ds(start, CH + L)], idx_buf)

            cmin = jnp.min(idx_buf[pl.ds(0, L)])
            cmax = jnp.max(idx_buf[pl.ds(CH - L, L)])
            overlap = (cmax >= base) & (cmin < base + HALF)

            @pl.when(overlap)
            def _():
                pltpu.sync_copy(sval_hbm.at[pl.ds(start, CH)], x_buf)

                def vec_body(i, c3):
                    kv = idx_buf[pl.ds(i * L, L)]
                    nv = idx_buf[pl.ds(i * L + 1, L)]
                    xv = x_buf[pl.ds(i * L, L)]
                    local = kv - base
                    m = (local >= 0) & (local < HALF) & (kv != nv)
                    safe = jnp.where(m, local, 0)
                    plsc.store_scatter(out_buf, [safe], xv, mask=m)
                    return c3

                lax.fori_loop(0, CH // L, vec_body, 0)

            return carry2

        lax.fori_loop(0, N // CH, chunk_body, 0)
        pltpu.sync_copy(out_buf, out_hbm.at[p, pl.ds(h * HALF, HALF)])
        return carry

    lax.fori_loop(0, TASKS_PER_WORKER, task_body, 0)


_unpool = functools.partial(
    pl.kernel,
    out_type=jax.ShapeDtypeStruct((P, HW), jnp.float32),
    mesh=plsc.VectorSubcoreMesh(core_axis_name="c", subcore_axis_name="s"),
    compiler_params=pltpu.CompilerParams(needs_layout_passes=False),
    scratch_types=[
        pltpu.VMEM((HALF,), jnp.float32),
        pltpu.VMEM((CH + L,), jnp.int32),
        pltpu.VMEM((CH,), jnp.float32),
    ],
)(_unpool_body)


def kernel(x, indices):
    b, c, h, w = x.shape
    plane = jnp.arange(b * c, dtype=jnp.int32)[:, None] * HW
    gidx = (indices.reshape(b * c, h * w) + plane).reshape(-1)
    s_idx, s_val = lax.sort((gidx, x.reshape(-1)), dimension=0,
                            is_stable=False, num_keys=1)
    out = _unpool(s_idx, s_val)
    return out.reshape(b, c, 2 * h, 2 * w)


# speedup vs baseline: 4.0651x; 1.0014x over previous
import functools

import jax
import jax.numpy as jnp
from jax import lax
from jax.experimental import pallas as pl
from jax.experimental.pallas import tpu as pltpu
from jax.experimental.pallas import tpu_sc as plsc

P = 384
N = 36864
HW = 147456
HALF = HW // 2
CH = 9216
N_WORKERS = 32
TASKS_PER_WORKER = (P * 2) // N_WORKERS
L = 16
BV = 36


def _unpool_body(sidx_hbm, sval_hbm, out_hbm, out_buf, idx_buf, x_buf):
    wid = lax.axis_index("s") * 2 + lax.axis_index("c")

    def task_body(t, carry):
        tid = wid * TASKS_PER_WORKER + t
        p = tid // 2
        h = tid % 2
        base = p * HW + h * HALF

        def zero_body(i, c):
            out_buf[pl.ds(i * L, L)] = jnp.zeros((L,), jnp.float32)
            return c

        lax.fori_loop(0, HALF // L, zero_body, 0)

        def chunk_body(c, carry2):
            start = p * N + c * CH
            is_last = start + CH >= P * N

            @pl.when(is_last)
            def _():
                pltpu.sync_copy(sidx_hbm.at[pl.ds(start, CH)],
                                idx_buf.at[pl.ds(0, CH)])
                idx_buf[pl.ds(CH, L)] = jnp.full((L,), -1, jnp.int32)

            @pl.when(jnp.logical_not(is_last))
            def _():
                pltpu.sync_copy(sidx_hbm.at[pl.ds(start, CH + L)], idx_buf)

            cmin = jnp.min(idx_buf[pl.ds(0, L)])
            cmax = jnp.max(idx_buf[pl.ds(CH - L, L)])
            overlap = (cmax >= base) & (cmin < base + HALF)

            @pl.when(overlap)
            def _():
                pltpu.sync_copy(sval_hbm.at[pl.ds(start, CH)], x_buf)

                def blk_body(bi, c4):
                    bmin = jnp.min(idx_buf[pl.ds(bi * BV * L, L)])
                    bmax = jnp.max(idx_buf[pl.ds((bi + 1) * BV * L - L, L)])
                    bover = (bmax >= base) & (bmin < base + HALF)

                    @pl.when(bover)
                    def _():
                        def vec_body(i, c3):
                            kv = idx_buf[pl.ds(i * L, L)]
                            nv = idx_buf[pl.ds(i * L + 1, L)]
                            xv = x_buf[pl.ds(i * L, L)]
                            local = kv - base
                            m = (local >= 0) & (local < HALF) & (kv != nv)
                            safe = jnp.where(m, local, 0)
                            plsc.store_scatter(out_buf, [safe], xv, mask=m)
                            return c3

                        lax.fori_loop(bi * BV, (bi + 1) * BV, vec_body, 0)

                    return c4

                lax.fori_loop(0, CH // L // BV, blk_body, 0)

            return carry2

        lax.fori_loop(0, N // CH, chunk_body, 0)
        pltpu.sync_copy(out_buf, out_hbm.at[p, pl.ds(h * HALF, HALF)])
        return carry

    lax.fori_loop(0, TASKS_PER_WORKER, task_body, 0)


_unpool = functools.partial(
    pl.kernel,
    out_type=jax.ShapeDtypeStruct((P, HW), jnp.float32),
    mesh=plsc.VectorSubcoreMesh(core_axis_name="c", subcore_axis_name="s"),
    compiler_params=pltpu.CompilerParams(needs_layout_passes=False),
    scratch_types=[
        pltpu.VMEM((HALF,), jnp.float32),
        pltpu.VMEM((CH + L,), jnp.int32),
        pltpu.VMEM((CH,), jnp.float32),
    ],
)(_unpool_body)


def kernel(x, indices):
    b, c, h, w = x.shape
    plane = jnp.arange(b * c, dtype=jnp.int32)[:, None] * HW
    gidx = (indices.reshape(b * c, h * w) + plane).reshape(-1)
    s_idx, s_val = lax.sort((gidx, x.reshape(-1)), dimension=0,
                            is_stable=False, num_keys=1)
    out = _unpool(s_idx, s_val)
    return out.reshape(b, c, 2 * h, 2 * w)
